# Initial kernel scaffold; baseline (speedup 1.0000x reference)
import jax
import jax.numpy as jnp
from jax.experimental import pallas as pl


def kernel(detections, masks, annotations, masks_target):
    raise NotImplementedError("write your pallas kernel here")


# speedup vs baseline: 3.5955x; 3.5955x over previous
import functools

import jax
import jax.numpy as jnp
from jax import lax
from jax.experimental import pallas as pl
from jax.experimental.pallas import tpu as pltpu
from jax.experimental.pallas import tpu_sc as plsc

IOU_THRESHOLD = 0.5
EPS = 1e-7

L = 16
NC = 2
NS = 16
NW = NC * NS
N_DET = 1024
K_ANN = 32
MH = MW = 28
IMG = 512
WIN_H = 136
WIN_W = 144
PER_W = N_DET // NW
N_PIX = MH * MW
R_VECS = N_PIX // L
BLK = MH * MW * K_ANN


def _sc_partials(det_t, masks_flat, ann_t, mt3):
    mesh = plsc.VectorSubcoreMesh(core_axis_name="c", subcore_axis_name="s")

    @functools.partial(
        pl.kernel,
        out_type=jax.ShapeDtypeStruct((NW, L), jnp.float32),
        mesh=mesh,
        compiler_params=pltpu.CompilerParams(
            use_tc_tiling_on_sc=False, needs_layout_passes=False),
        scratch_types=[
            pltpu.VMEM((PER_W, L), jnp.float32),
            pltpu.VMEM((4, K_ANN), jnp.float32),
            pltpu.VMEM((BLK,), jnp.float32),
            pltpu.VMEM((WIN_H, WIN_W), jnp.float32),
            pltpu.VMEM((L,), jnp.float32),
        ],
    )
    def k(det_hbm, masks_hbm, ann_hbm, mt_hbm, out_hbm, det_v, ann_v, mblk, win, pbuf):
        wid = lax.axis_index("s") * NC + lax.axis_index("c")
        base = wid * PER_W
        pltpu.sync_copy(det_hbm.at[pl.ds(base, PER_W), :], det_v)
        pltpu.sync_copy(ann_hbm, ann_v)

        lane = lax.iota(jnp.int32, L)
        bx1a = ann_v[0, pl.ds(0, L)]
        bx1b = ann_v[0, pl.ds(L, L)]
        by1a = ann_v[1, pl.ds(0, L)]
        by1b = ann_v[1, pl.ds(L, L)]
        bx2a = ann_v[2, pl.ds(0, L)]
        bx2b = ann_v[2, pl.ds(L, L)]
        by2a = ann_v[3, pl.ds(0, L)]
        by2b = ann_v[3, pl.ds(L, L)]
        areaa = (bx2a - bx1a) * (by2a - by1a)
        areab = (bx2b - bx1b) * (by2b - by1b)

        def det_body(d, carry):
            tot, cnt = carry
            dv = det_v[d]
            ax1 = dv[0]
            ay1 = dv[1]
            ax2 = dv[2]
            ay2 = dv[3]
            area_d = (ax2 - ax1) * (ay2 - ay1)

            def iou_half(bx1, by1, bx2, by2, area_b):
                iw = jnp.maximum(jnp.minimum(ax2, bx2) - jnp.maximum(ax1, bx1), 0.0)
                ih = jnp.maximum(jnp.minimum(ay2, by2) - jnp.maximum(ay1, by1), 0.0)
                ua = jnp.maximum(area_d + area_b - iw * ih, EPS)
                return iw * ih / ua

            iou0 = iou_half(bx1a, by1a, bx2a, by2a, areaa)
            iou1 = iou_half(bx1b, by1b, bx2b, by2b, areab)
            mx = jnp.maximum(jnp.max(iou0), jnp.max(iou1))
            eq0 = iou0 == mx
            eq1 = iou1 == mx
            pc0 = plsc.all_reduce_population_count(eq0)
            f0 = plsc.all_reduce_ffs(eq0)
            f1 = plsc.all_reduce_ffs(eq1)
            c = jnp.max(jnp.where(pc0 > 0, f0, f1 + L))
            sel = jnp.where(mx >= IOU_THRESHOLD, 1.0, 0.0)

            scale = float(IMG - 1) / float(IMG)
            a_y = ay1 * scale
            a_x = ax1 * scale
            s_y = (ay2 - ay1) * (scale / float(MH - 1))
            s_x = (ax2 - ax1) * (scale / float(MW - 1))
            y_start = jnp.minimum(a_y.astype(jnp.int32), IMG - WIN_H)
            x_start = jnp.minimum(
                (a_x.astype(jnp.int32) // L) * L, IMG - WIN_W)

            pltpu.sync_copy(
                mt_hbm.at[c, pl.ds(y_start, WIN_H), pl.ds(x_start, WIN_W)], win)
            pltpu.sync_copy(masks_hbm.at[base + d], mblk)

            def r_body(r, acc):
                p = lane + r * L
                i = p // MW
                j = p - i * MW
                ys = a_y + i.astype(jnp.float32) * s_y
                xs = a_x + j.astype(jnp.float32) * s_x
                y0 = ys.astype(jnp.int32)
                x0 = xs.astype(jnp.int32)
                wy = ys - y0.astype(jnp.float32)
                wx = xs - x0.astype(jnp.float32)
                ry0 = jnp.minimum(y0, IMG - 1) - y_start
                ry1 = jnp.minimum(y0 + 1, IMG - 1) - y_start
                rx0 = jnp.minimum(x0, IMG - 1) - x_start
                rx1 = jnp.minimum(x0 + 1, IMG - 1) - x_start
                g00 = plsc.load_gather(win, [ry0, rx0])
                g01 = plsc.load_gather(win, [ry0, rx1])
                g10 = plsc.load_gather(win, [ry1, rx0])
                g11 = plsc.load_gather(win, [ry1, rx1])
                top = g00 + wx * (g01 - g00)
                bot = g10 + wx * (g11 - g10)
                crop = top + wy * (bot - top)
                mg = plsc.load_gather(mblk, [p * K_ANN + c])
                return acc + jnp.abs(mg - crop)

            acc = lax.fori_loop(0, R_VECS, r_body, jnp.zeros((L,), jnp.float32))
            return tot + jnp.sum(acc) * sel, cnt + sel

        tot, cnt = lax.fori_loop(
            0, PER_W, det_body,
            (jnp.float32(0.0), jnp.float32(0.0)))
        pbuf[...] = jnp.where(lane == 0, tot, jnp.where(lane == 1, cnt, 0.0))
        pltpu.sync_copy(pbuf, out_hbm.at[wid])

    return k(det_t, masks_flat, ann_t, mt3)


def _finish(p_ref, o_ref):
    x = p_ref[...]
    s = jnp.sum(x[:, 0:1])
    c = jnp.sum(x[:, 1:2])
    o_ref[...] = jnp.full((1, 1), s / jnp.maximum(c * float(N_PIX), 1.0),
                          jnp.float32)


def kernel(detections, masks, annotations, masks_target):
    det_t = jnp.pad(detections[0], ((0, 0), (0, L - 4)))
    ann_t = jnp.transpose(annotations[0], (1, 0))
    masks_flat = masks.reshape(N_DET, BLK)
    mt3 = masks_target.reshape(K_ANN, IMG, IMG)
    partials = _sc_partials(det_t, masks_flat, ann_t, mt3)
    out = pl.pallas_call(
        _finish,
        out_shape=jax.ShapeDtypeStruct((1, 1), jnp.float32),
    )(partials)
    return out[0, 0]
